# baseline (speedup 1.0000x reference)
import functools

import jax
import jax.numpy as jnp
from jax.experimental import pallas as pl
from jax.experimental.pallas import tpu as pltpu
from jax.experimental.pallas import tpu_sc as plsc

N_NODES = 10000
N_EDGES = 320000
HIDDEN = 128
EDGE_DIM = 16
OUT_DIM = 2 * HIDDEN
BN_EPS = 1e-5

NUM_CORES = 2
NUM_SUBCORES = 16
NUM_WORKERS = NUM_CORES * NUM_SUBCORES

N_CHUNKS_E = 4
CHUNK_E = N_EDGES // N_CHUNKS_E

GATHER_W = 128
EDGE_TILE = 1600
TILES_PER_CHUNK = CHUNK_E // EDGE_TILE

SCHUNK = 128
SC_CHUNKS = CHUNK_E // SCHUNK
N_NODES_PAD = 10240
ROWS_PER_SUBCORE = N_NODES_PAD // NUM_SUBCORES
ZROWS = 128


def _sc_mesh():
    return plsc.VectorSubcoreMesh(core_axis_name="core",
                                  subcore_axis_name="subcore")


def _sc_gather(node_feats, src_idx, dst_idx):
    out_t = jax.ShapeDtypeStruct((CHUNK_E, HIDDEN), node_feats.dtype)

    @functools.partial(
        pl.kernel, out_type=(out_t, out_t), mesh=_sc_mesh(),
        scratch_types=[pltpu.SemaphoreType.DMA, pltpu.SemaphoreType.DMA],
    )
    def k(nf_hbm, si_hbm, di_hbm, os_hbm, od_hbm, sem_s, sem_d):
        def body(si_v, di_v, os_v, od_v):
            d1 = pltpu.async_copy(nf_hbm.at[si_v.at[0]], os_v, sem_s)
            d2 = pltpu.async_copy(nf_hbm.at[di_v.at[0]], od_v, sem_d)
            d1.wait()
            d2.wait()

        pltpu.emit_pipeline(
            body,
            grid=(CHUNK_E // GATHER_W,),
            in_specs=[
                pl.BlockSpec((1, GATHER_W), lambda i: (0, i)),
                pl.BlockSpec((1, GATHER_W), lambda i: (0, i)),
            ],
            out_specs=[
                pl.BlockSpec((GATHER_W, HIDDEN), lambda i: (i, 0)),
                pl.BlockSpec((GATHER_W, HIDDEN), lambda i: (i, 0)),
            ],
            core_axis_name=("core", "subcore"),
            dimension_semantics=(pltpu.PARALLEL,),
        )(si_hbm, di_hbm, os_hbm, od_hbm)

    return k(node_feats, src_idx, dst_idx)


def _pass_a(src_rows, dst_rows, edge_feats, w12t, w3t, b_row, chunk):
    base = chunk * TILES_PER_CHUNK

    def body(src_ref, dst_ref, ef_ref, w12_ref, w3_ref, b_ref,
             z_ref, s1_ref, s2_ref):
        x = jnp.concatenate([src_ref[...], dst_ref[...]], axis=1)
        z = jnp.dot(x.astype(jnp.bfloat16), w12_ref[...],
                    preferred_element_type=jnp.float32)
        z = z + jnp.dot(ef_ref[...], w3_ref[...],
                        preferred_element_type=jnp.float32)
        z = z + b_ref[...]
        z_ref[...] = z.astype(jnp.bfloat16)
        s1_ref[...] = jnp.sum(z, axis=0, keepdims=True)[None]
        s2_ref[...] = jnp.sum(z * z, axis=0, keepdims=True)[None]

    return pl.pallas_call(
        body,
        grid=(TILES_PER_CHUNK,),
        in_specs=[
            pl.BlockSpec((EDGE_TILE, HIDDEN), lambda i: (i, 0)),
            pl.BlockSpec((EDGE_TILE, HIDDEN), lambda i: (i, 0)),
            pl.BlockSpec((EDGE_TILE, EDGE_DIM), lambda i: (base + i, 0)),
            pl.BlockSpec((2 * HIDDEN, OUT_DIM), lambda i: (0, 0)),
            pl.BlockSpec((EDGE_DIM, OUT_DIM), lambda i: (0, 0)),
            pl.BlockSpec((1, OUT_DIM), lambda i: (0, 0)),
        ],
        out_specs=[
            pl.BlockSpec((EDGE_TILE, OUT_DIM), lambda i: (i, 0)),
            pl.BlockSpec((1, 1, OUT_DIM), lambda i: (i, 0, 0)),
            pl.BlockSpec((1, 1, OUT_DIM), lambda i: (i, 0, 0)),
        ],
        out_shape=[
            jax.ShapeDtypeStruct((CHUNK_E, OUT_DIM), jnp.bfloat16),
            jax.ShapeDtypeStruct((TILES_PER_CHUNK, 1, OUT_DIM), jnp.float32),
            jax.ShapeDtypeStruct((TILES_PER_CHUNK, 1, OUT_DIM), jnp.float32),
        ],
    )(src_rows, dst_rows, edge_feats, w12t, w3t, b_row)


def _finalize(s1s, s2s, gamma_row, beta_row):
    def body(*refs):
        s_refs = refs[:N_CHUNKS_E]
        q_refs = refs[N_CHUNKS_E:2 * N_CHUNKS_E]
        g_ref, be_ref, sc_ref, sh_ref = refs[2 * N_CHUNKS_E:]
        inv_n = jnp.float32(1.0 / N_EDGES)
        s1 = sum(jnp.sum(r[...], axis=0) for r in s_refs)
        s2 = sum(jnp.sum(r[...], axis=0) for r in q_refs)
        mean = s1 * inv_n
        var = s2 * inv_n - mean * mean
        scale = g_ref[...] * jax.lax.rsqrt(var + BN_EPS)
        sc_ref[...] = scale
        sh_ref[...] = be_ref[...] - mean * scale

    return pl.pallas_call(
        body,
        out_shape=[
            jax.ShapeDtypeStruct((1, OUT_DIM), jnp.float32),
            jax.ShapeDtypeStruct((1, OUT_DIM), jnp.float32),
        ],
    )(*s1s, *s2s, gamma_row, beta_row)


def _pass_b(z_bf, scale, shift):
    def body(z_ref, sc_ref, sh_ref, m_ref):
        zn = z_ref[...].astype(jnp.float32) * sc_ref[...] + sh_ref[...]
        gate = zn[:, :HIDDEN]
        msg = zn[:, HIDDEN:]
        m_ref[...] = jax.nn.sigmoid(gate) * jax.nn.softplus(msg)

    return pl.pallas_call(
        body,
        grid=(TILES_PER_CHUNK,),
        in_specs=[
            pl.BlockSpec((EDGE_TILE, OUT_DIM), lambda i: (i, 0)),
            pl.BlockSpec((1, OUT_DIM), lambda i: (0, 0)),
            pl.BlockSpec((1, OUT_DIM), lambda i: (0, 0)),
        ],
        out_specs=pl.BlockSpec((EDGE_TILE, HIDDEN), lambda i: (i, 0)),
        out_shape=jax.ShapeDtypeStruct((CHUNK_E, HIDDEN), jnp.float32),
    )(z_bf, scale, shift)


SC1_CPW = SC_CHUNKS // NUM_WORKERS
SC1_REM = SC_CHUNKS - SC1_CPW * NUM_WORKERS


def _sc_scatter1(m, di):
    @functools.partial(
        pl.kernel,
        out_type=jax.ShapeDtypeStruct((NUM_CORES, N_NODES_PAD, HIDDEN),
                                      jnp.float32),
        mesh=_sc_mesh(),
        scratch_types=[
            pltpu.VMEM_SHARED((N_NODES_PAD, HIDDEN), jnp.float32),
            pltpu.VMEM((SCHUNK, HIDDEN), jnp.float32),
            pltpu.VMEM((1, SCHUNK), jnp.int32),
            pltpu.VMEM((ZROWS, HIDDEN), jnp.float32),
        ],
    )
    def k(m_hbm, di_hbm, out_hbm, acc_sh, m_v, idx_v, z_v):
        cid = jax.lax.axis_index("core")
        sid = jax.lax.axis_index("subcore")

        zvec = jnp.zeros((16,), jnp.float32)

        @pl.loop(0, ZROWS)
        def _(r):
            @pl.loop(0, HIDDEN, step=16)
            def _(c0):
                z_v[r, pl.ds(c0, 16)] = zvec

        my_rows = sid * ROWS_PER_SUBCORE

        @pl.loop(0, ROWS_PER_SUBCORE, step=ZROWS)
        def _(r0):
            pltpu.sync_copy(z_v, acc_sh.at[pl.ds(my_rows + r0, ZROWS)])

        plsc.subcore_barrier()

        wid = sid * NUM_CORES + cid

        def do_chunk(c):
            pltpu.sync_copy(di_hbm.at[c], idx_v.at[0])
            pltpu.sync_copy(m_hbm.at[pl.ds(c * SCHUNK, SCHUNK)], m_v)
            pltpu.sync_copy(m_v, acc_sh.at[idx_v.at[0]], add=True)

        @pl.loop(0, SC1_CPW)
        def _(j):
            do_chunk(wid * SC1_CPW + j)

        @pl.when(wid < SC1_REM)
        def _():
            do_chunk(NUM_WORKERS * SC1_CPW + wid)

        plsc.subcore_barrier()
        pltpu.sync_copy(
            acc_sh.at[pl.ds(my_rows, ROWS_PER_SUBCORE)],
            out_hbm.at[cid, pl.ds(my_rows, ROWS_PER_SUBCORE)])

    return k(m, di)


def _final(node_feats, partials):
    tile = 1000

    def body(*refs):
        nf_ref = refs[0]
        p_refs = refs[1:-1]
        o_ref = refs[-1]
        acc = nf_ref[...]
        for p in p_refs:
            acc = acc + p[0] + p[1]
        o_ref[...] = jax.nn.softplus(acc)

    p_spec = pl.BlockSpec((NUM_CORES, tile, HIDDEN), lambda i: (0, i, 0))
    return pl.pallas_call(
        body,
        grid=(N_NODES // tile,),
        in_specs=[pl.BlockSpec((tile, HIDDEN), lambda i: (i, 0))]
        + [p_spec] * len(partials),
        out_specs=pl.BlockSpec((tile, HIDDEN), lambda i: (i, 0)),
        out_shape=jax.ShapeDtypeStruct((N_NODES, HIDDEN), jnp.float32),
    )(node_feats, *partials)


def kernel(node_feats, edge_feats, edge_index, W, b, gamma, beta):
    edge_index = edge_index.astype(jnp.int32)
    src_idx = edge_index[0].reshape(1, N_EDGES)
    dst_idx = edge_index[1].reshape(1, N_EDGES)

    w12t = W[:, :2 * HIDDEN].T.astype(jnp.bfloat16)
    w3t = W[:, 2 * HIDDEN:].T.astype(jnp.bfloat16)
    b_row = b.reshape(1, OUT_DIM)
    gamma_row = gamma.reshape(1, OUT_DIM)
    beta_row = beta.reshape(1, OUT_DIM)

    ef16 = edge_feats.astype(jnp.bfloat16)

    zs, s1s, s2s = [], [], []
    for c in range(N_CHUNKS_E):
        lo = c * CHUNK_E
        src_c = jax.lax.dynamic_slice(src_idx, (0, lo), (1, CHUNK_E))
        dst_c = jax.lax.dynamic_slice(dst_idx, (0, lo), (1, CHUNK_E))
        sr, dr = _sc_gather(node_feats, src_c, dst_c)
        z_c, s1_c, s2_c = _pass_a(sr, dr, ef16, w12t, w3t, b_row, c)
        zs.append(z_c)
        s1s.append(s1_c)
        s2s.append(s2_c)

    scale, shift = _finalize(s1s, s2s, gamma_row, beta_row)

    msgs = [_pass_b(z_c, scale, shift) for z_c in zs]
    dis = [dst_idx[0, c * CHUNK_E:(c + 1) * CHUNK_E].reshape(SC_CHUNKS,
                                                            SCHUNK)
           for c in range(N_CHUNKS_E)]

    partials = [_sc_scatter1(msgs[c], dis[c]) for c in range(N_CHUNKS_E)]
    return _final(node_feats, partials)


# speedup vs baseline: 1.0133x; 1.0133x over previous
import functools

import jax
import jax.numpy as jnp
from jax.experimental import pallas as pl
from jax.experimental.pallas import tpu as pltpu
from jax.experimental.pallas import tpu_sc as plsc

N_NODES = 10000
N_EDGES = 320000
HIDDEN = 128
EDGE_DIM = 16
OUT_DIM = 2 * HIDDEN
BN_EPS = 1e-5

NUM_CORES = 2
NUM_SUBCORES = 16
NUM_WORKERS = NUM_CORES * NUM_SUBCORES

N_CHUNKS_E = 4
CHUNK_E = N_EDGES // N_CHUNKS_E

GATHER_W = 128
EDGE_TILE = 1600
TILES_PER_CHUNK = CHUNK_E // EDGE_TILE

SCHUNK = 128
SC_CHUNKS = CHUNK_E // SCHUNK
N_NODES_PAD = 10240
ROWS_PER_SUBCORE = N_NODES_PAD // NUM_SUBCORES
ZROWS = 128


def _sc_mesh():
    return plsc.VectorSubcoreMesh(core_axis_name="core",
                                  subcore_axis_name="subcore")


def _sc_gather(node_feats, src_idx, dst_idx):
    out_t = jax.ShapeDtypeStruct((CHUNK_E, HIDDEN), node_feats.dtype)

    @functools.partial(pl.kernel, out_type=(out_t, out_t), mesh=_sc_mesh())
    def k(nf_hbm, si_hbm, di_hbm, os_hbm, od_hbm):
        def body(si_v, di_v, os_v, od_v):
            pltpu.sync_copy(nf_hbm.at[si_v.at[0]], os_v)
            pltpu.sync_copy(nf_hbm.at[di_v.at[0]], od_v)

        pltpu.emit_pipeline(
            body,
            grid=(CHUNK_E // GATHER_W,),
            in_specs=[
                pl.BlockSpec((1, GATHER_W), lambda i: (0, i)),
                pl.BlockSpec((1, GATHER_W), lambda i: (0, i)),
            ],
            out_specs=[
                pl.BlockSpec((GATHER_W, HIDDEN), lambda i: (i, 0)),
                pl.BlockSpec((GATHER_W, HIDDEN), lambda i: (i, 0)),
            ],
            core_axis_name=("core", "subcore"),
            dimension_semantics=(pltpu.PARALLEL,),
        )(si_hbm, di_hbm, os_hbm, od_hbm)

    return k(node_feats, src_idx, dst_idx)


def _pass_a(src_rows, dst_rows, edge_feats, w12t, w3t, b_row, chunk):
    base = chunk * TILES_PER_CHUNK

    def body(src_ref, dst_ref, ef_ref, w12_ref, w3_ref, b_ref,
             z_ref, s1_ref, s2_ref):
        x = jnp.concatenate([src_ref[...], dst_ref[...]], axis=1)
        z = jnp.dot(x.astype(jnp.bfloat16), w12_ref[...],
                    preferred_element_type=jnp.float32)
        z = z + jnp.dot(ef_ref[...], w3_ref[...],
                        preferred_element_type=jnp.float32)
        z = z + b_ref[...]
        z_ref[...] = z.astype(jnp.bfloat16)
        s1_ref[...] = jnp.sum(z, axis=0, keepdims=True)[None]
        s2_ref[...] = jnp.sum(z * z, axis=0, keepdims=True)[None]

    return pl.pallas_call(
        body,
        grid=(TILES_PER_CHUNK,),
        in_specs=[
            pl.BlockSpec((EDGE_TILE, HIDDEN), lambda i: (i, 0)),
            pl.BlockSpec((EDGE_TILE, HIDDEN), lambda i: (i, 0)),
            pl.BlockSpec((EDGE_TILE, EDGE_DIM), lambda i: (base + i, 0)),
            pl.BlockSpec((2 * HIDDEN, OUT_DIM), lambda i: (0, 0)),
            pl.BlockSpec((EDGE_DIM, OUT_DIM), lambda i: (0, 0)),
            pl.BlockSpec((1, OUT_DIM), lambda i: (0, 0)),
        ],
        out_specs=[
            pl.BlockSpec((EDGE_TILE, OUT_DIM), lambda i: (i, 0)),
            pl.BlockSpec((1, 1, OUT_DIM), lambda i: (i, 0, 0)),
            pl.BlockSpec((1, 1, OUT_DIM), lambda i: (i, 0, 0)),
        ],
        out_shape=[
            jax.ShapeDtypeStruct((CHUNK_E, OUT_DIM), jnp.bfloat16),
            jax.ShapeDtypeStruct((TILES_PER_CHUNK, 1, OUT_DIM), jnp.float32),
            jax.ShapeDtypeStruct((TILES_PER_CHUNK, 1, OUT_DIM), jnp.float32),
        ],
    )(src_rows, dst_rows, edge_feats, w12t, w3t, b_row)


def _finalize(s1s, s2s, gamma_row, beta_row):
    def body(*refs):
        s_refs = refs[:N_CHUNKS_E]
        q_refs = refs[N_CHUNKS_E:2 * N_CHUNKS_E]
        g_ref, be_ref, sc_ref, sh_ref = refs[2 * N_CHUNKS_E:]
        inv_n = jnp.float32(1.0 / N_EDGES)
        s1 = sum(jnp.sum(r[...], axis=0) for r in s_refs)
        s2 = sum(jnp.sum(r[...], axis=0) for r in q_refs)
        mean = s1 * inv_n
        var = s2 * inv_n - mean * mean
        scale = g_ref[...] * jax.lax.rsqrt(var + BN_EPS)
        sc_ref[...] = scale
        sh_ref[...] = be_ref[...] - mean * scale

    return pl.pallas_call(
        body,
        out_shape=[
            jax.ShapeDtypeStruct((1, OUT_DIM), jnp.float32),
            jax.ShapeDtypeStruct((1, OUT_DIM), jnp.float32),
        ],
    )(*s1s, *s2s, gamma_row, beta_row)


def _pass_b(z_bf, scale, shift):
    def body(z_ref, sc_ref, sh_ref, m_ref):
        zn = z_ref[...].astype(jnp.float32) * sc_ref[...] + sh_ref[...]
        gate = zn[:, :HIDDEN]
        msg = zn[:, HIDDEN:]
        m_ref[...] = jax.nn.sigmoid(gate) * jax.nn.softplus(msg)

    return pl.pallas_call(
        body,
        grid=(TILES_PER_CHUNK,),
        in_specs=[
            pl.BlockSpec((EDGE_TILE, OUT_DIM), lambda i: (i, 0)),
            pl.BlockSpec((1, OUT_DIM), lambda i: (0, 0)),
            pl.BlockSpec((1, OUT_DIM), lambda i: (0, 0)),
        ],
        out_specs=pl.BlockSpec((EDGE_TILE, HIDDEN), lambda i: (i, 0)),
        out_shape=jax.ShapeDtypeStruct((CHUNK_E, HIDDEN), jnp.float32),
    )(z_bf, scale, shift)


SC1_CPW = SC_CHUNKS // NUM_WORKERS
SC1_REM = SC_CHUNKS - SC1_CPW * NUM_WORKERS


def _sc_scatter1(m, di):
    @functools.partial(
        pl.kernel,
        out_type=jax.ShapeDtypeStruct((NUM_CORES, N_NODES_PAD, HIDDEN),
                                      jnp.float32),
        mesh=_sc_mesh(),
        scratch_types=[
            pltpu.VMEM_SHARED((N_NODES_PAD, HIDDEN), jnp.float32),
            pltpu.VMEM((SCHUNK, HIDDEN), jnp.float32),
            pltpu.VMEM((1, SCHUNK), jnp.int32),
            pltpu.VMEM((ZROWS, HIDDEN), jnp.float32),
        ],
    )
    def k(m_hbm, di_hbm, out_hbm, acc_sh, m_v, idx_v, z_v):
        cid = jax.lax.axis_index("core")
        sid = jax.lax.axis_index("subcore")

        zvec = jnp.zeros((16,), jnp.float32)

        @pl.loop(0, ZROWS)
        def _(r):
            @pl.loop(0, HIDDEN, step=16)
            def _(c0):
                z_v[r, pl.ds(c0, 16)] = zvec

        my_rows = sid * ROWS_PER_SUBCORE

        @pl.loop(0, ROWS_PER_SUBCORE, step=ZROWS)
        def _(r0):
            pltpu.sync_copy(z_v, acc_sh.at[pl.ds(my_rows + r0, ZROWS)])

        plsc.subcore_barrier()

        wid = sid * NUM_CORES + cid

        def do_chunk(c):
            pltpu.sync_copy(di_hbm.at[c], idx_v.at[0])
            pltpu.sync_copy(m_hbm.at[pl.ds(c * SCHUNK, SCHUNK)], m_v)
            pltpu.sync_copy(m_v, acc_sh.at[idx_v.at[0]], add=True)

        @pl.loop(0, SC1_CPW)
        def _(j):
            do_chunk(wid * SC1_CPW + j)

        @pl.when(wid < SC1_REM)
        def _():
            do_chunk(NUM_WORKERS * SC1_CPW + wid)

        plsc.subcore_barrier()
        pltpu.sync_copy(
            acc_sh.at[pl.ds(my_rows, ROWS_PER_SUBCORE)],
            out_hbm.at[cid, pl.ds(my_rows, ROWS_PER_SUBCORE)])

    return k(m, di)


def _final(node_feats, partials):
    tile = 1000

    def body(*refs):
        nf_ref = refs[0]
        p_refs = refs[1:-1]
        o_ref = refs[-1]
        acc = nf_ref[...]
        for p in p_refs:
            acc = acc + p[0] + p[1]
        o_ref[...] = jax.nn.softplus(acc)

    p_spec = pl.BlockSpec((NUM_CORES, tile, HIDDEN), lambda i: (0, i, 0))
    return pl.pallas_call(
        body,
        grid=(N_NODES // tile,),
        in_specs=[pl.BlockSpec((tile, HIDDEN), lambda i: (i, 0))]
        + [p_spec] * len(partials),
        out_specs=pl.BlockSpec((tile, HIDDEN), lambda i: (i, 0)),
        out_shape=jax.ShapeDtypeStruct((N_NODES, HIDDEN), jnp.float32),
    )(node_feats, *partials)


def kernel(node_feats, edge_feats, edge_index, W, b, gamma, beta):
    edge_index = edge_index.astype(jnp.int32)
    src_idx = edge_index[0].reshape(1, N_EDGES)
    dst_idx = edge_index[1].reshape(1, N_EDGES)

    w12t = W[:, :2 * HIDDEN].T.astype(jnp.bfloat16)
    w3t = W[:, 2 * HIDDEN:].T.astype(jnp.bfloat16)
    b_row = b.reshape(1, OUT_DIM)
    gamma_row = gamma.reshape(1, OUT_DIM)
    beta_row = beta.reshape(1, OUT_DIM)

    ef16 = edge_feats.astype(jnp.bfloat16)

    zs, s1s, s2s = [], [], []
    for c in range(N_CHUNKS_E):
        lo = c * CHUNK_E
        src_c = jax.lax.dynamic_slice(src_idx, (0, lo), (1, CHUNK_E))
        dst_c = jax.lax.dynamic_slice(dst_idx, (0, lo), (1, CHUNK_E))
        sr, dr = _sc_gather(node_feats, src_c, dst_c)
        z_c, s1_c, s2_c = _pass_a(sr, dr, ef16, w12t, w3t, b_row, c)
        zs.append(z_c)
        s1s.append(s1_c)
        s2s.append(s2_c)

    scale, shift = _finalize(s1s, s2s, gamma_row, beta_row)

    msgs = [_pass_b(z_c, scale, shift) for z_c in zs]
    dis = [dst_idx[0, c * CHUNK_E:(c + 1) * CHUNK_E].reshape(SC_CHUNKS,
                                                            SCHUNK)
           for c in range(N_CHUNKS_E)]

    partials = [_sc_scatter1(msgs[c], dis[c]) for c in range(N_CHUNKS_E)]
    return _final(node_feats, partials)
